# Initial kernel scaffold; baseline (speedup 1.0000x reference)
import jax
import jax.numpy as jnp
from jax.experimental import pallas as pl


def kernel(x, w1, b1, w2, b2):
    raise NotImplementedError("write your pallas kernel here")


# speedup vs baseline: 1.2218x; 1.2218x over previous
from functools import partial

import jax
import jax.numpy as jnp
from jax.experimental import pallas as pl
from jax.experimental.pallas import tpu as pltpu

_VMEM_LIMIT_BYTES = 96 * 1024 * 1024
_BLOCK_BUDGET = 28 * 1024 * 1024


def _ca_kernel(x_ref, w1t_ref, b1_ref, w2_ref, b2_ref, o_ref, *, inv_hw):
    x = x_ref[...]
    pooled = jnp.sum(x, axis=2, keepdims=True) * inv_hw
    w1t = w1t_ref[...][None]
    h = jnp.sum(pooled * w1t, axis=1, keepdims=True)
    h = jnp.maximum(h + b1_ref[...][None], 0.0)
    w2 = w2_ref[...][None]
    s = jnp.sum(w2 * h, axis=2, keepdims=True)
    s = jax.nn.sigmoid(s + b2_ref[...][None])
    o_ref[...] = x * s


def kernel(x, w1, b1, w2, b2):
    N, C, H, W = x.shape
    Cr = w1.shape[0]
    HW = H * W
    inv_hw = 1.0 / HW
    itemsize = jnp.dtype(x.dtype).itemsize

    B = 1
    for cand in (16, 8, 4, 2):
        if N % cand == 0 and 4 * cand * C * HW * itemsize <= _BLOCK_BUDGET:
            B = cand
            break

    x_flat = x.reshape(N, C, HW)
    w1t = jnp.asarray(w1, jnp.float32).T
    b1r = jnp.asarray(b1, jnp.float32).reshape(1, Cr)
    w2r = jnp.asarray(w2, jnp.float32)
    b2r = jnp.asarray(b2, jnp.float32).reshape(C, 1)

    out = pl.pallas_call(
        partial(_ca_kernel, inv_hw=inv_hw),
        out_shape=jax.ShapeDtypeStruct((N, C, HW), x.dtype),
        grid=(N // B,),
        in_specs=[
            pl.BlockSpec((B, C, HW), lambda n: (n, 0, 0)),
            pl.BlockSpec((C, Cr), lambda n: (0, 0)),
            pl.BlockSpec((1, Cr), lambda n: (0, 0)),
            pl.BlockSpec((C, Cr), lambda n: (0, 0)),
            pl.BlockSpec((C, 1), lambda n: (0, 0)),
        ],
        out_specs=pl.BlockSpec((B, C, HW), lambda n: (n, 0, 0)),
        compiler_params=pltpu.CompilerParams(
            dimension_semantics=("parallel",),
            vmem_limit_bytes=_VMEM_LIMIT_BYTES),
    )(x_flat, w1t, b1r, w2r, b2r)
    return out.reshape(N, C, H, W)
